# baseline (speedup 1.0000x reference)
import functools

import jax
import jax.numpy as jnp
from jax import lax
from jax.experimental import pallas as pl
from jax.experimental.pallas import tpu as pltpu
from jax.experimental.pallas import tpu_sc as plsc

B = 16384
EMBED_K = 64
FEAT_DIM = 512

_NC = 2
_NS = 16
_NW = _NC * _NS
_BPW = B // _NW

_FCHUNK = 64
_NFCHUNK = _BPW // _FCHUNK
_TCHUNK = 128
_NTCHUNK = _BPW // _TCHUNK


def _sc_gather_f(items, F_feat, off, bh):
    mesh = plsc.VectorSubcoreMesh(core_axis_name="c", subcore_axis_name="s")

    nbuf = 3
    bpw = bh // _NW
    nchunk = bpw // _FCHUNK

    @functools.partial(
        pl.kernel,
        mesh=mesh,
        out_type=jax.ShapeDtypeStruct((bh, FEAT_DIM), jnp.float32),
        scratch_types=[
            pltpu.VMEM((bpw,), jnp.int32),
            pltpu.VMEM((nbuf, _FCHUNK, FEAT_DIM), jnp.float32),
            [pltpu.SemaphoreType.DMA] * nbuf,
            [pltpu.SemaphoreType.DMA] * nbuf,
        ],
    )
    def k(items_hbm, f_hbm, effe_out, iidx_v, rows_v, gsems, wsems):
        _BPW = bpw
        _NFCHUNK = nchunk
        wid = lax.axis_index("s") * _NC + lax.axis_index("c")
        base = wid * _BPW
        pltpu.sync_copy(items_hbm.at[pl.ds(off + base, _BPW)], iidx_v)
        gcopies = [None] * nbuf
        wcopies = [None] * nbuf
        for c in range(min(nbuf, _NFCHUNK)):
            gcopies[c] = pltpu.async_copy(
                f_hbm.at[iidx_v.at[pl.ds(c * _FCHUNK, _FCHUNK)]],
                rows_v.at[c], gsems[c])
        for c in range(_NFCHUNK):
            s = c % nbuf
            gcopies[s].wait()
            wcopies[s] = pltpu.async_copy(
                rows_v.at[s], effe_out.at[pl.ds(base + c * _FCHUNK, _FCHUNK)],
                wsems[s])
            nxt = c + nbuf
            if nxt < _NFCHUNK:
                wcopies[s].wait()
                gcopies[s] = pltpu.async_copy(
                    f_hbm.at[iidx_v.at[pl.ds(nxt * _FCHUNK, _FCHUNK)]],
                    rows_v.at[s], gsems[s])
        for c in range(max(_NFCHUNK - nbuf, 0), _NFCHUNK):
            wcopies[c % nbuf].wait()

    return k(items, F_feat)


def _sc_gather_tu(upairs, Tu_pairs):
    mesh = plsc.VectorSubcoreMesh(core_axis_name="c", subcore_axis_name="s")

    @functools.partial(
        pl.kernel,
        mesh=mesh,
        out_type=jax.ShapeDtypeStruct((B, 2 * EMBED_K), jnp.float32),
        scratch_types=[
            pltpu.VMEM((_BPW,), jnp.int32),
            pltpu.VMEM((2, _TCHUNK, 2 * EMBED_K), jnp.float32),
            pltpu.SemaphoreType.DMA,
            pltpu.SemaphoreType.DMA,
        ],
    )
    def k(upairs_hbm, tu_hbm, theta_out, uidx_v, rows_v, sem0, sem1):
        wid = lax.axis_index("s") * _NC + lax.axis_index("c")
        base = wid * _BPW
        pltpu.sync_copy(upairs_hbm.at[pl.ds(base, _BPW)], uidx_v)
        sems = (sem0, sem1)
        copies = [None, None]
        copies[0] = pltpu.async_copy(
            tu_hbm.at[uidx_v.at[pl.ds(0, _TCHUNK)]], rows_v.at[0], sems[0])
        for c in range(_NTCHUNK):
            nxt = c + 1
            if nxt < _NTCHUNK:
                copies[nxt % 2] = pltpu.async_copy(
                    tu_hbm.at[uidx_v.at[pl.ds(nxt * _TCHUNK, _TCHUNK)]],
                    rows_v.at[nxt % 2], sems[nxt % 2])
            copies[c % 2].wait()
            pltpu.sync_copy(rows_v.at[c % 2],
                            theta_out.at[pl.ds(base + c * _TCHUNK, _TCHUNK)])

    return k(upairs, Tu_pairs)


_PLANES = 8192
_PSHIFT = _PLANES.bit_length() - 1
_HALF = _PLANES // 2


def _tc_pairs(TuT):
    NU = TuT.shape[1]
    nb = (NU + _PLANES - 1) // _PLANES
    half = _HALF

    def body(x_ref, o_ref):
        y = jnp.transpose(x_ref[...])
        o_ref[...] = jnp.concatenate(
            [lax.slice(y, (0, 0), (half, EMBED_K)),
             lax.slice(y, (half, 0), (_PLANES, EMBED_K))], axis=1)

    return pl.pallas_call(
        body,
        grid=(nb,),
        in_specs=[pl.BlockSpec((EMBED_K, _PLANES), lambda i: (0, i))],
        out_specs=pl.BlockSpec((half, 2 * EMBED_K), lambda i: (i, 0)),
        out_shape=jax.ShapeDtypeStruct((nb * half, 2 * EMBED_K), jnp.float32),
    )(TuT)


_TBLK = 2048


def _proj_body(effe_ref, w_ref, b_ref, projT_ref):
    proj = jnp.dot(effe_ref[...], w_ref[...],
                   preferred_element_type=jnp.float32)
    projT = jnp.transpose(proj) + b_ref[...]
    ones = jnp.ones((1, EMBED_K), jnp.float32)
    sT = jnp.dot(ones, projT * projT, preferred_element_type=jnp.float32)
    invT = jnp.where(sT > 1e-24, lax.rsqrt(sT), 1e12)
    projT_ref[...] = projT * invT


def _proj_alias_body(effe_ref, w_ref, b_ref, _pin_ref, projT_ref):
    _proj_body(effe_ref, w_ref, b_ref, projT_ref)


def _tc_proj(effe_h, proj_W, proj_b1, h, prev=None):
    nblk = effe_h.shape[0] // _TBLK
    base = h * nblk
    in_specs = [
        pl.BlockSpec((_TBLK, FEAT_DIM), lambda i: (i, 0)),
        pl.BlockSpec((FEAT_DIM, EMBED_K), lambda i: (0, 0)),
        pl.BlockSpec((EMBED_K, 1), lambda i: (0, 0)),
    ]
    args = [effe_h, proj_W, proj_b1]
    kwargs = {}
    body = _proj_body
    if prev is not None:
        in_specs.append(pl.BlockSpec(memory_space=pl.ANY))
        args.append(prev)
        kwargs["input_output_aliases"] = {3: 0}
        body = _proj_alias_body
    return pl.pallas_call(
        body,
        grid=(nblk,),
        in_specs=in_specs,
        out_specs=pl.BlockSpec((EMBED_K, _TBLK), lambda i: (0, i + base)),
        out_shape=jax.ShapeDtypeStruct((EMBED_K, B), jnp.float32),
        **kwargs,
    )(*args)


def _fin_body(guT_ref, giT_ref, theta2_ref, upar_ref, projT_ref,
              xui_ref, thetaT_ref, gouT_ref, goiT_ref):
    par = (lax.shift_right_logical(upar_ref[...], _PSHIFT - 1) & 1)[:, None]
    theta = jnp.where(par == 1, theta2_ref[:, EMBED_K:], theta2_ref[:, :EMBED_K])
    thetaT = jnp.transpose(theta)
    guT = guT_ref[...]
    giT = giT_ref[...]
    ones = jnp.ones((1, EMBED_K), jnp.float32)
    xui = jnp.dot(ones, guT * giT + thetaT * projT_ref[...],
                  preferred_element_type=jnp.float32)
    xui_ref[...] = xui[None]
    thetaT_ref[...] = thetaT
    gouT_ref[...] = guT
    goiT_ref[...] = giT


def _tc_fin(guT, giT, theta2, users, projT):
    return pl.pallas_call(
        _fin_body,
        grid=(B // _TBLK,),
        in_specs=[
            pl.BlockSpec((EMBED_K, _TBLK), lambda i: (0, i)),
            pl.BlockSpec((EMBED_K, _TBLK), lambda i: (0, i)),
            pl.BlockSpec((_TBLK, 2 * EMBED_K), lambda i: (i, 0)),
            pl.BlockSpec((_TBLK,), lambda i: (i,)),
            pl.BlockSpec((EMBED_K, _TBLK), lambda i: (0, i)),
        ],
        out_specs=[
            pl.BlockSpec((1, 1, _TBLK), lambda i: (i, 0, 0)),
            pl.BlockSpec((EMBED_K, _TBLK), lambda i: (0, i)),
            pl.BlockSpec((EMBED_K, _TBLK), lambda i: (0, i)),
            pl.BlockSpec((EMBED_K, _TBLK), lambda i: (0, i)),
        ],
        out_shape=[
            jax.ShapeDtypeStruct((B // _TBLK, 1, _TBLK), jnp.float32),
            jax.ShapeDtypeStruct((EMBED_K, B), jnp.float32),
            jax.ShapeDtypeStruct((EMBED_K, B), jnp.float32),
            jax.ShapeDtypeStruct((EMBED_K, B), jnp.float32),
        ],
    )(guT, giT, theta2, users, projT)


def kernel(gu, gi, users, items, Tu_weight, F_feat, proj_W, proj_b):
    effe_0 = _sc_gather_f(items, F_feat, 0, B // 2)
    effe_1 = _sc_gather_f(items, F_feat, B // 2, B // 2)
    Tu_pairs = _tc_pairs(jnp.transpose(Tu_weight))
    zidx = (lax.shift_right_logical(users, _PSHIFT) * _HALF
            + (users & (_HALF - 1)))
    zidx, effe_1 = lax.optimization_barrier((zidx, effe_1))
    theta2 = _sc_gather_tu(zidx, Tu_pairs)
    b1 = jnp.reshape(proj_b, (EMBED_K, 1))
    p0 = _tc_proj(effe_0, proj_W, b1, 0)
    projT = _tc_proj(effe_1, proj_W, b1, 1, prev=p0)
    xui2, thetaT, gammaT_u, gammaT_i = _tc_fin(
        jnp.transpose(gu), jnp.transpose(gi), theta2, users, projT)
    return (jnp.reshape(xui2, (B,)), jnp.transpose(gammaT_u),
            jnp.transpose(gammaT_i), jnp.transpose(thetaT),
            jnp.transpose(projT))


# speedup vs baseline: 1.0447x; 1.0447x over previous
import functools

import jax
import jax.numpy as jnp
from jax import lax
from jax.experimental import pallas as pl
from jax.experimental.pallas import tpu as pltpu
from jax.experimental.pallas import tpu_sc as plsc

B = 16384
EMBED_K = 64
FEAT_DIM = 512

_NC = 2
_NS = 16
_NW = _NC * _NS
_BPW = B // _NW

_FCHUNK = 64
_NFCHUNK = _BPW // _FCHUNK
_TCHUNK = 128
_NTCHUNK = _BPW // _TCHUNK


def _sc_gather_f(items, F_feat, off, bh):
    mesh = plsc.VectorSubcoreMesh(core_axis_name="c", subcore_axis_name="s")

    nbuf = 3
    bpw = bh // _NW
    nchunk = bpw // _FCHUNK

    @functools.partial(
        pl.kernel,
        mesh=mesh,
        out_type=jax.ShapeDtypeStruct((bh, FEAT_DIM), jnp.float32),
        scratch_types=[
            pltpu.VMEM((bpw,), jnp.int32),
            pltpu.VMEM((nbuf, _FCHUNK, FEAT_DIM), jnp.float32),
            [pltpu.SemaphoreType.DMA] * nbuf,
            [pltpu.SemaphoreType.DMA] * nbuf,
        ],
    )
    def k(items_hbm, f_hbm, effe_out, iidx_v, rows_v, gsems, wsems):
        _BPW = bpw
        _NFCHUNK = nchunk
        wid = lax.axis_index("s") * _NC + lax.axis_index("c")
        base = wid * _BPW
        pltpu.sync_copy(items_hbm.at[pl.ds(off + base, _BPW)], iidx_v)
        gcopies = [None] * nbuf
        wcopies = [None] * nbuf
        for c in range(min(nbuf, _NFCHUNK)):
            gcopies[c] = pltpu.async_copy(
                f_hbm.at[iidx_v.at[pl.ds(c * _FCHUNK, _FCHUNK)]],
                rows_v.at[c], gsems[c])
        for c in range(_NFCHUNK):
            s = c % nbuf
            gcopies[s].wait()
            wcopies[s] = pltpu.async_copy(
                rows_v.at[s], effe_out.at[pl.ds(base + c * _FCHUNK, _FCHUNK)],
                wsems[s])
            nxt = c + nbuf
            if nxt < _NFCHUNK:
                wcopies[s].wait()
                gcopies[s] = pltpu.async_copy(
                    f_hbm.at[iidx_v.at[pl.ds(nxt * _FCHUNK, _FCHUNK)]],
                    rows_v.at[s], gsems[s])
        for c in range(max(_NFCHUNK - nbuf, 0), _NFCHUNK):
            wcopies[c % nbuf].wait()

    return k(items, F_feat)


def _sc_gather_tu(upairs, Tu_pairs):
    mesh = plsc.VectorSubcoreMesh(core_axis_name="c", subcore_axis_name="s")

    @functools.partial(
        pl.kernel,
        mesh=mesh,
        out_type=jax.ShapeDtypeStruct((B, 2 * EMBED_K), jnp.float32),
        scratch_types=[
            pltpu.VMEM((_BPW,), jnp.int32),
            pltpu.VMEM((2, _TCHUNK, 2 * EMBED_K), jnp.float32),
            pltpu.SemaphoreType.DMA,
            pltpu.SemaphoreType.DMA,
        ],
    )
    def k(upairs_hbm, tu_hbm, theta_out, uidx_v, rows_v, sem0, sem1):
        wid = lax.axis_index("s") * _NC + lax.axis_index("c")
        base = wid * _BPW
        pltpu.sync_copy(upairs_hbm.at[pl.ds(base, _BPW)], uidx_v)
        sems = (sem0, sem1)
        copies = [None, None]
        copies[0] = pltpu.async_copy(
            tu_hbm.at[uidx_v.at[pl.ds(0, _TCHUNK)]], rows_v.at[0], sems[0])
        for c in range(_NTCHUNK):
            nxt = c + 1
            if nxt < _NTCHUNK:
                copies[nxt % 2] = pltpu.async_copy(
                    tu_hbm.at[uidx_v.at[pl.ds(nxt * _TCHUNK, _TCHUNK)]],
                    rows_v.at[nxt % 2], sems[nxt % 2])
            copies[c % 2].wait()
            pltpu.sync_copy(rows_v.at[c % 2],
                            theta_out.at[pl.ds(base + c * _TCHUNK, _TCHUNK)])

    return k(upairs, Tu_pairs)


_PLANES = 8192
_PSHIFT = _PLANES.bit_length() - 1
_HALF = _PLANES // 2


def _tc_pairs(TuT):
    NU = TuT.shape[1]
    nb = (NU + _PLANES - 1) // _PLANES
    half = _HALF

    def body(x_ref, o_ref):
        y = jnp.transpose(x_ref[...])
        o_ref[...] = jnp.concatenate(
            [lax.slice(y, (0, 0), (half, EMBED_K)),
             lax.slice(y, (half, 0), (_PLANES, EMBED_K))], axis=1)

    return pl.pallas_call(
        body,
        grid=(nb,),
        in_specs=[pl.BlockSpec((EMBED_K, _PLANES), lambda i: (0, i))],
        out_specs=pl.BlockSpec((half, 2 * EMBED_K), lambda i: (i, 0)),
        out_shape=jax.ShapeDtypeStruct((nb * half, 2 * EMBED_K), jnp.float32),
    )(TuT)


_TBLK = 2048


def _tc_gammas(guT, giT):
    def body(a_ref, b_ref, oa_ref, ob_ref):
        oa_ref[...] = a_ref[...]
        ob_ref[...] = b_ref[...]

    return pl.pallas_call(
        body,
        grid=(B // _TBLK,),
        in_specs=[
            pl.BlockSpec((EMBED_K, _TBLK), lambda i: (0, i)),
            pl.BlockSpec((EMBED_K, _TBLK), lambda i: (0, i)),
        ],
        out_specs=[
            pl.BlockSpec((EMBED_K, _TBLK), lambda i: (0, i)),
            pl.BlockSpec((EMBED_K, _TBLK), lambda i: (0, i)),
        ],
        out_shape=[
            jax.ShapeDtypeStruct((EMBED_K, B), jnp.float32),
            jax.ShapeDtypeStruct((EMBED_K, B), jnp.float32),
        ],
    )(guT, giT)


def _tc_dense_body(guT_ref, giT_ref, theta2_ref, upar_ref, effe_ref, w_ref,
                   b_ref, xui_ref, thetaT_ref, projT_ref):
    proj = jnp.dot(effe_ref[...], w_ref[...],
                   preferred_element_type=jnp.float32)
    par = (lax.shift_right_logical(upar_ref[...], _PSHIFT - 1) & 1)[:, None]
    theta = jnp.where(par == 1, theta2_ref[:, EMBED_K:], theta2_ref[:, :EMBED_K])
    projT = jnp.transpose(proj) + b_ref[...]
    thetaT = jnp.transpose(theta)
    ones = jnp.ones((1, EMBED_K), jnp.float32)
    sT = jnp.dot(ones, projT * projT, preferred_element_type=jnp.float32)
    invT = jnp.where(sT > 1e-24, lax.rsqrt(sT), 1e12)
    proj_iT = projT * invT
    guT = guT_ref[...]
    giT = giT_ref[...]
    xui = jnp.dot(ones, guT * giT + thetaT * proj_iT,
                  preferred_element_type=jnp.float32)
    xui_ref[...] = xui[None]
    thetaT_ref[...] = thetaT
    projT_ref[...] = proj_iT


def _tc_dense(guT, giT, theta2, users, effe_i, proj_W, proj_b):
    return pl.pallas_call(
        _tc_dense_body,
        grid=(B // _TBLK,),
        in_specs=[
            pl.BlockSpec((EMBED_K, _TBLK), lambda i: (0, i)),
            pl.BlockSpec((EMBED_K, _TBLK), lambda i: (0, i)),
            pl.BlockSpec((_TBLK, 2 * EMBED_K), lambda i: (i, 0)),
            pl.BlockSpec((_TBLK,), lambda i: (i,)),
            pl.BlockSpec((_TBLK, FEAT_DIM), lambda i: (i, 0)),
            pl.BlockSpec((FEAT_DIM, EMBED_K), lambda i: (0, 0)),
            pl.BlockSpec((EMBED_K, 1), lambda i: (0, 0)),
        ],
        out_specs=[
            pl.BlockSpec((1, 1, _TBLK), lambda i: (i, 0, 0)),
            pl.BlockSpec((EMBED_K, _TBLK), lambda i: (0, i)),
            pl.BlockSpec((EMBED_K, _TBLK), lambda i: (0, i)),
        ],
        out_shape=[
            jax.ShapeDtypeStruct((B // _TBLK, 1, _TBLK), jnp.float32),
            jax.ShapeDtypeStruct((EMBED_K, B), jnp.float32),
            jax.ShapeDtypeStruct((EMBED_K, B), jnp.float32),
        ],
    )(guT, giT, theta2, users, effe_i, proj_W,
      jnp.reshape(proj_b, (EMBED_K, 1)))


def kernel(gu, gi, users, items, Tu_weight, F_feat, proj_W, proj_b):
    effe_i = _sc_gather_f(items, F_feat, 0, B)
    Tu_pairs = _tc_pairs(jnp.transpose(Tu_weight))
    zidx = (lax.shift_right_logical(users, _PSHIFT) * _HALF
            + (users & (_HALF - 1)))
    zidx, effe_i = lax.optimization_barrier((zidx, effe_i))
    theta2 = _sc_gather_tu(zidx, Tu_pairs)
    gammaT_u, gammaT_i = _tc_gammas(jnp.transpose(gu), jnp.transpose(gi))
    xui2, thetaT, projT = _tc_dense(
        jnp.transpose(gu), jnp.transpose(gi), theta2, users, effe_i,
        proj_W, proj_b)
    return (jnp.reshape(xui2, (B,)), jnp.transpose(gammaT_u),
            jnp.transpose(gammaT_i), jnp.transpose(thetaT),
            jnp.transpose(projT))
